# Initial kernel scaffold; baseline (speedup 1.0000x reference)
import jax
import jax.numpy as jnp
from jax.experimental import pallas as pl


def kernel(x, lips_W1, lips_b1, lips_W2, lips_b2, lips_W3, lips_b3, lips_W4, lips_b4, exp_W1, exp_b1, exp_W2, exp_b2, exp_W3, exp_b3, exp_W4, exp_b4, rest_W1, rest_b1, rest_W2, rest_b2, rest_W3, rest_b3, rest_W4, rest_b4, rot_scale_W1, rot_scale_b1, rot_scale_W2, rot_scale_b2, rot_scale_W3, rot_scale_b3, rot_scale_W4, rot_scale_b4, mean, std):
    raise NotImplementedError("write your pallas kernel here")


# speedup vs baseline: 1.3968x; 1.3968x over previous
import functools

import jax
import jax.numpy as jnp
import numpy as np
from jax.experimental import pallas as pl
from jax.experimental.pallas import tpu as pltpu

_L = 5
_H = 512

_HIGH = jax.lax.Precision.HIGHEST


def _dot(a, b):
    return jax.lax.dot(a, b, precision=_HIGH, preferred_element_type=jnp.float32)


def _bdot(a, b):
    return jax.lax.dot(a.astype(jnp.bfloat16), b, preferred_element_type=jnp.float32)


def _fsq_kernel(x_ref, w1a_ref, b1a_ref, w1r_ref, b1r_ref, w2_ref, b2_ref,
                w3_ref, b3_ref, w4a_ref, b4a_ref, w4r_ref, b4r_ref,
                wi_ref, offs_ref, std_ref, mean_ref, out_ref, idx_ref):
    xb = x_ref[...]
    a = jnp.concatenate(
        [xb[:, 60:75], xb[:, 12:60], xb[:, 0:9], xb[:, 138:139]], axis=1)
    r = jnp.concatenate([xb[:, 9:12], xb[:, 75:138], xb[:, 139:205]], axis=1)

    h1a = jnp.maximum(_bdot(a, w1a_ref[...]) + b1a_ref[...], 0.0)
    h1r = jnp.maximum(_bdot(r, w1r_ref[...]) + b1r_ref[...], 0.0)
    h1 = jnp.concatenate([h1a, h1r], axis=1)

    z = _bdot(h1, w2_ref[...]) + b2_ref[...]
    t1 = 0.25541281188299536
    t2 = 0.9729550745276566
    codes = ((z > -t2).astype(jnp.float32) + (z > -t1).astype(jnp.float32)
             + (z > t1).astype(jnp.float32) + (z > t2).astype(jnp.float32))
    zq = codes - 2.0

    idxf = _dot(codes, wi_ref[...]) + offs_ref[...]
    idx_ref[...] = idxf.astype(jnp.int32)

    zqd = zq * 0.5
    h2 = jnp.maximum(_bdot(zqd, w3_ref[...]) + b3_ref[...], 0.0)
    reca = _bdot(h2[:, :1536], w4a_ref[...]) + b4a_ref[...]
    recr = _bdot(h2[:, 1536:], w4r_ref[...]) + b4r_ref[...]

    out = jnp.concatenate([
        reca[:, 63:72],
        recr[:, 0:3],
        reca[:, 15:63],
        reca[:, 0:15],
        recr[:, 3:66],
        reca[:, 72:73],
        recr[:, 66:69],
        recr[:, 69:132],
    ], axis=1)
    out_ref[...] = out * std_ref[...] + mean_ref[...]


def _block_diag(blocks):
    rows = sum(b.shape[0] for b in blocks)
    cols = sum(b.shape[1] for b in blocks)
    out = jnp.zeros((rows, cols), dtype=blocks[0].dtype)
    r = c = 0
    for b in blocks:
        out = jax.lax.dynamic_update_slice(out, b, (r, c))
        r += b.shape[0]
        c += b.shape[1]
    return out


@functools.partial(jax.jit, static_argnums=())
def kernel(x, lips_W1, lips_b1, lips_W2, lips_b2, lips_W3, lips_b3, lips_W4, lips_b4,
           exp_W1, exp_b1, exp_W2, exp_b2, exp_W3, exp_b3, exp_W4, exp_b4,
           rest_W1, rest_b1, rest_W2, rest_b2, rest_W3, rest_b3, rest_W4, rest_b4,
           rot_scale_W1, rot_scale_b1, rot_scale_W2, rot_scale_b2,
           rot_scale_W3, rot_scale_b3, rot_scale_W4, rot_scale_b4,
           mean, std):
    Bt, Ft, C = x.shape
    n_rows = Bt * Ft
    x2 = x.reshape(n_rows, C)

    w1a = _block_diag([lips_W1, exp_W1, rot_scale_W1])
    b1a = jnp.concatenate([lips_b1, exp_b1, rot_scale_b1])[None, :]
    w1r = rest_W1
    b1r = rest_b1[None, :]
    w2 = _block_diag([lips_W2, exp_W2, rot_scale_W2, rest_W2])
    b2 = jnp.concatenate([lips_b2, exp_b2, rot_scale_b2, rest_b2])[None, :]
    w3 = _block_diag([lips_W3, exp_W3, rot_scale_W3, rest_W3])
    b3 = jnp.concatenate([lips_b3, exp_b3, rot_scale_b3, rest_b3])[None, :]
    w4a = _block_diag([lips_W4, exp_W4, rot_scale_W4])
    b4a = jnp.concatenate([lips_b4, exp_b4, rot_scale_b4])[None, :]
    w4r = rest_W4
    b4r = rest_b4[None, :]

    pw6 = _L ** np.arange(6, dtype=np.float32)
    pw8 = _L ** np.arange(8, dtype=np.float32)
    pw4 = _L ** np.arange(4, dtype=np.float32)
    wi = np.zeros((24, 4), dtype=np.float32)
    wi[0:6, 0] = pw6
    wi[6:12, 1] = pw6
    wi[16:24, 2] = pw8
    wi[12:16, 3] = pw4
    offs = np.array([[0.0, _L**6, 2 * _L**6, 2 * _L**6 + _L**8]], dtype=np.float32)
    wi = jnp.asarray(wi)
    offs = jnp.asarray(offs)

    bf = jnp.bfloat16
    w1a = w1a.astype(bf)
    w1r = w1r.astype(bf)
    w2 = w2.astype(bf)
    w3 = w3.astype(bf)
    w4a = w4a.astype(bf)
    w4r = w4r.astype(bf)

    std2 = std[None, :]
    mean2 = mean[None, :]

    blk = 1024
    grid = (n_rows // blk,)

    def _rep(shape):
        return pl.BlockSpec(shape, lambda i: (0,) * len(shape))

    out, idx = pl.pallas_call(
        _fsq_kernel,
        grid=grid,
        in_specs=[
            pl.BlockSpec((blk, C), lambda i: (i, 0)),
            _rep(w1a.shape), _rep(b1a.shape), _rep(w1r.shape), _rep(b1r.shape),
            _rep(w2.shape), _rep(b2.shape), _rep(w3.shape), _rep(b3.shape),
            _rep(w4a.shape), _rep(b4a.shape), _rep(w4r.shape), _rep(b4r.shape),
            _rep(wi.shape), _rep(offs.shape), _rep(std2.shape), _rep(mean2.shape),
        ],
        out_specs=[
            pl.BlockSpec((blk, C), lambda i: (i, 0)),
            pl.BlockSpec((blk, 4), lambda i: (i, 0)),
        ],
        out_shape=[
            jax.ShapeDtypeStruct((n_rows, C), jnp.float32),
            jax.ShapeDtypeStruct((n_rows, 4), jnp.int32),
        ],
        compiler_params=pltpu.CompilerParams(
            dimension_semantics=("parallel",),
        ),
    )(x2, w1a, b1a, w1r, b1r, w2, b2, w3, b3, w4a, b4a, w4r, b4r,
      wi, offs, std2, mean2)

    out = out.reshape(Bt, Ft, C)
    codes_stacked = idx.T.reshape(4, Bt, Ft)
    return out, codes_stacked
